# Initial kernel scaffold; baseline (speedup 1.0000x reference)
import jax
import jax.numpy as jnp
from jax.experimental import pallas as pl


def kernel(X, cb0, cb1, cb2, cb3, return_dist):
    raise NotImplementedError("write your pallas kernel here")


# speedup vs baseline: 1.4210x; 1.4210x over previous
import jax
import jax.numpy as jnp
from jax.experimental import pallas as pl

_T = 1024


def _rvq_body(x_ref, c0, c1, c2, c3, idx_ref, dist_ref):
    r = x_ref[:]
    cbs = (c0, c1, c2, c3)
    K = c0.shape[0]
    for s in range(4):
        W = cbs[s][:]
        cn = jnp.sum(W * W, axis=1, keepdims=True)
        S = jax.lax.dot_general(
            r, W, dimension_numbers=(((1,), (1,)), ((), ())),
            preferred_element_type=jnp.float32)
        rn = jnp.sum(r * r, axis=1, keepdims=True)
        sq = (rn + cn.T) - 2.0 * S
        m = jnp.min(sq, axis=1, keepdims=True)
        iota = jax.lax.broadcasted_iota(jnp.int32, sq.shape, 1)
        i = jnp.min(jnp.where(sq == m, iota, K), axis=1, keepdims=True)
        idx_ref[:, s : s + 1] = i
        dist_ref[:, s : s + 1] = jnp.sqrt(jnp.maximum(m, 1e-12))
        if s < 3:
            onehot = (iota == i).astype(jnp.float32)
            w_sel = jax.lax.dot_general(
                onehot, W, dimension_numbers=(((1,), (0,)), ((), ())),
                preferred_element_type=jnp.float32,
                precision=jax.lax.Precision.HIGHEST)
            r = r - w_sel


def kernel(X, cb0, cb1, cb2, cb3, return_dist):
    N, D = X.shape
    K = cb0.shape[0]
    grid = N // _T
    idx, dist = pl.pallas_call(
        _rvq_body,
        grid=(grid,),
        in_specs=[pl.BlockSpec((_T, D), lambda i: (i, 0))]
        + [pl.BlockSpec((K, D), lambda i: (0, 0))] * 4,
        out_specs=[pl.BlockSpec((_T, 4), lambda i: (i, 0)),
                   pl.BlockSpec((_T, 4), lambda i: (i, 0))],
        out_shape=[jax.ShapeDtypeStruct((N, 4), jnp.int32),
                   jax.ShapeDtypeStruct((N, 4), jnp.float32)],
    )(X, cb0, cb1, cb2, cb3)
    gate = jnp.asarray(return_dist, jnp.float32)
    return idx, dist * gate


# speedup vs baseline: 2.6459x; 1.8620x over previous
import jax
import jax.numpy as jnp
from jax.experimental import pallas as pl

_T = 1024


def _split2(W):
    h1 = W.astype(jnp.bfloat16)
    h2 = (W - h1.astype(jnp.float32)).astype(jnp.bfloat16)
    return h1, h2


def _rvq_body(x_ref, c0, c1, c2, c3, n0, n1, n2, n3, *rest):
    (h10, h20, h11, h21, h12, h22, idx_ref, dist_ref) = rest
    r = x_ref[:]
    cbs = (c0, c1, c2, c3)
    cns = (n0, n1, n2, n3)
    his = ((h10, h20), (h11, h21), (h12, h22), None)
    K = c0.shape[0]
    rn = jnp.sum(r * r, axis=1, keepdims=True)
    idxs = []
    dsqs = []
    for s in range(4):
        Wm2 = cbs[s][:]
        Sm2 = jax.lax.dot_general(
            r, Wm2, dimension_numbers=(((1,), (1,)), ((), ())),
            preferred_element_type=jnp.float32)
        sc = cns[s][:] + Sm2
        m = jnp.min(sc, axis=1, keepdims=True)
        iota_f = jax.lax.broadcasted_iota(
            jnp.int32, sc.shape, 1).astype(jnp.float32)
        i_f = jnp.min(jnp.where(sc == m, iota_f, jnp.float32(K)),
                      axis=1, keepdims=True)
        idxs.append(i_f.astype(jnp.int32))
        rn = rn + m
        dsqs.append(rn)
        if s < 3:
            onehot = (iota_f == i_f).astype(jnp.bfloat16)
            g = None
            for h in his[s]:
                part = jax.lax.dot_general(
                    onehot, h[:], dimension_numbers=(((1,), (0,)), ((), ())),
                    preferred_element_type=jnp.float32)
                g = part if g is None else g + part
            r = r - g
    idx_ref[:] = jnp.concatenate(idxs, axis=1)
    dist_ref[:] = jnp.sqrt(
        jnp.maximum(jnp.concatenate(dsqs, axis=1), 1e-12))


def kernel(X, cb0, cb1, cb2, cb3, return_dist):
    N, D = X.shape
    K = cb0.shape[0]
    grid = N // _T
    cbs = (cb0, cb1, cb2, cb3)
    cbs_m2 = tuple(-2.0 * W for W in cbs)
    cns = tuple(jnp.sum(W * W, axis=1)[None, :] for W in cbs)
    splits = _split2(cb0) + _split2(cb1) + _split2(cb2)
    wspec = pl.BlockSpec((K, D), lambda i: (0, 0))
    nspec = pl.BlockSpec((1, K), lambda i: (0, 0))
    idx, dist = pl.pallas_call(
        _rvq_body,
        grid=(grid,),
        in_specs=[pl.BlockSpec((_T, D), lambda i: (i, 0))]
        + [wspec] * 4 + [nspec] * 4 + [wspec] * 6,
        out_specs=[pl.BlockSpec((_T, 4), lambda i: (i, 0)),
                   pl.BlockSpec((_T, 4), lambda i: (i, 0))],
        out_shape=[jax.ShapeDtypeStruct((N, 4), jnp.int32),
                   jax.ShapeDtypeStruct((N, 4), jnp.float32)],
    )(X, *cbs_m2, *cns, *splits)
    gate = jnp.asarray(return_dist, jnp.float32)
    return idx, dist * gate


# speedup vs baseline: 3.8221x; 1.4446x over previous
import jax
import jax.numpy as jnp
from jax.experimental import pallas as pl

_T = 2048


def _split2(W):
    h1 = W.astype(jnp.bfloat16)
    h2 = (W - h1.astype(jnp.float32)).astype(jnp.bfloat16)
    return jnp.concatenate([h1, h2], axis=1)


def _rvq_body(x_ref, c0, c1, c2, c3, n0, n1, n2, n3,
              h0, h1, h2, idx_ref, dist_ref):
    x = x_ref[:]
    T, D = x.shape
    cbs = (c0, c1, c2, c3)
    cns = (n0, n1, n2, n3)
    his = (h0, h1, h2, None)
    K = c0.shape[0]
    iota_f = jax.lax.broadcasted_iota(
        jnp.int32, (T, K), 1).astype(jnp.float32)
    r = x
    rn = jnp.sum(r * r, axis=1, keepdims=True)
    idxs = []
    dsqs = []
    for s in range(4):
        Sm2 = jax.lax.dot_general(
            r, cbs[s][:], dimension_numbers=(((1,), (1,)), ((), ())),
            preferred_element_type=jnp.float32)
        sc = cns[s][:] + Sm2
        m = jnp.min(sc, axis=1, keepdims=True)
        i_f = jnp.min(jnp.where(sc == m, iota_f, jnp.float32(K)),
                      axis=1, keepdims=True)
        idxs.append(i_f.astype(jnp.int32))
        rn = rn + m
        dsqs.append(rn)
        if s < 3:
            onehot = (iota_f == i_f).astype(jnp.bfloat16)
            gg = jax.lax.dot_general(
                onehot, his[s][:], dimension_numbers=(((1,), (0,)), ((), ())),
                preferred_element_type=jnp.float32)
            r = r - (gg[:, :D] + gg[:, D:])
    idx_ref[:] = jnp.concatenate(idxs, axis=1)
    dist_ref[:] = jnp.sqrt(
        jnp.maximum(jnp.concatenate(dsqs, axis=1), 1e-12))


def kernel(X, cb0, cb1, cb2, cb3, return_dist):
    N, D = X.shape
    K = cb0.shape[0]
    grid = N // _T
    cbs = (cb0, cb1, cb2, cb3)
    cbs_m2 = tuple(-2.0 * W for W in cbs)
    cns = tuple(jnp.sum(W * W, axis=1)[None, :] for W in cbs)
    splits = tuple(_split2(W) for W in cbs[:3])
    wspec = pl.BlockSpec((K, D), lambda i: (0, 0))
    nspec = pl.BlockSpec((1, K), lambda i: (0, 0))
    hspec = pl.BlockSpec((K, 2 * D), lambda i: (0, 0))
    idx, dist = pl.pallas_call(
        _rvq_body,
        grid=(grid,),
        in_specs=[pl.BlockSpec((_T, D), lambda i: (i, 0))]
        + [wspec] * 4 + [nspec] * 4 + [hspec] * 3,
        out_specs=[pl.BlockSpec((_T, 4), lambda i: (i, 0)),
                   pl.BlockSpec((_T, 4), lambda i: (i, 0))],
        out_shape=[jax.ShapeDtypeStruct((N, 4), jnp.int32),
                   jax.ShapeDtypeStruct((N, 4), jnp.float32)],
    )(X, *cbs_m2, *cns, *splits)
    gate = jnp.asarray(return_dist, jnp.float32)
    return idx, dist * gate


# speedup vs baseline: 4.2653x; 1.1159x over previous
import jax
import jax.numpy as jnp
from jax.experimental import pallas as pl

_T = 4096
_CW = 128


def _split2(W):
    h1 = W.astype(jnp.bfloat16)
    h2 = (W - h1.astype(jnp.float32)).astype(jnp.bfloat16)
    return jnp.concatenate([h1, h2], axis=1)


def _aug(W):
    K, D = W.shape
    cn = jnp.sum(W * W, axis=1)
    c1 = cn.astype(jnp.bfloat16)
    rcl = cn - c1.astype(jnp.float32)
    c2 = rcl.astype(jnp.bfloat16)
    c3 = (rcl - c2.astype(jnp.float32)).astype(jnp.bfloat16)
    cols = [c.astype(jnp.float32)[:, None] for c in (c1, c2, c3)]
    pad = jnp.zeros((K, _CW - D - 3), jnp.float32)
    return jnp.concatenate([-2.0 * W] + cols + [pad], axis=1)


def _rvq_body(x_ref, c0, c1, c2, c3, h0, h1, h2, idx_ref, dist_ref):
    x = x_ref[:]
    T, D = x.shape
    cbs = (c0, c1, c2, c3)
    his = (h0, h1, h2, None)
    K = c0.shape[0]
    iota_f = jax.lax.broadcasted_iota(
        jnp.int32, (1, K), 1).astype(jnp.float32)
    onespad = jnp.concatenate(
        [jnp.ones((T, 3), jnp.float32),
         jnp.zeros((T, _CW - D - 3), jnp.float32)], axis=1)
    r = x
    rn = jnp.sum(r * r, axis=1, keepdims=True)
    idxs = []
    dsqs = []
    for s in range(4):
        ra = jnp.concatenate([r, onespad], axis=1)
        sc = jax.lax.dot_general(
            ra, cbs[s][:], dimension_numbers=(((1,), (1,)), ((), ())),
            preferred_element_type=jnp.float32)
        m = jnp.min(sc, axis=1, keepdims=True)
        i_f = jnp.min(jnp.where(sc == m, iota_f, jnp.float32(K)),
                      axis=1, keepdims=True)
        idxs.append(i_f.astype(jnp.int32))
        rn = rn + m
        dsqs.append(rn)
        if s < 3:
            onehot = (iota_f == i_f).astype(jnp.bfloat16)
            gg = jax.lax.dot_general(
                onehot, his[s][:], dimension_numbers=(((1,), (0,)), ((), ())),
                preferred_element_type=jnp.float32)
            r = r - (gg[:, :D] + gg[:, D:])
    idx_ref[:] = jnp.concatenate(idxs, axis=1)
    dist_ref[:] = jnp.sqrt(
        jnp.maximum(jnp.concatenate(dsqs, axis=1), 1e-12))


def kernel(X, cb0, cb1, cb2, cb3, return_dist):
    N, D = X.shape
    K = cb0.shape[0]
    grid = N // _T
    cbs = (cb0, cb1, cb2, cb3)
    augs = tuple(_aug(W) for W in cbs)
    splits = tuple(_split2(W) for W in cbs[:3])
    aspec = pl.BlockSpec((K, _CW), lambda i: (0, 0))
    hspec = pl.BlockSpec((K, 2 * D), lambda i: (0, 0))
    idx, dist = pl.pallas_call(
        _rvq_body,
        grid=(grid,),
        in_specs=[pl.BlockSpec((_T, D), lambda i: (i, 0))]
        + [aspec] * 4 + [hspec] * 3,
        out_specs=[pl.BlockSpec((_T, 4), lambda i: (i, 0)),
                   pl.BlockSpec((_T, 4), lambda i: (i, 0))],
        out_shape=[jax.ShapeDtypeStruct((N, 4), jnp.int32),
                   jax.ShapeDtypeStruct((N, 4), jnp.float32)],
    )(X, *augs, *splits)
    gate = jnp.asarray(return_dist, jnp.float32)
    return idx, dist * gate
